# baseline (speedup 1.0000x reference)
import functools

import jax
import jax.numpy as jnp
from jax.experimental import pallas as pl
from jax.experimental.pallas import tpu as pltpu

N_NODES = 50000
N_FEATS = 256
K = 256
ROW_BLOCK = 2000
N_BLOCKS = N_NODES // ROW_BLOCK
P_ROWS = 512
P_COLS = 128
N_PADDED = P_ROWS * P_COLS


def _score_kernel(f_ref, w_ref, m_ref, o_ref):
    w = w_ref[...]
    nrm = jnp.sqrt(jnp.sum(w * w))
    s = jnp.dot(f_ref[...], w, preferred_element_type=jnp.float32)
    s = s / nrm
    m = m_ref[...]
    o_ref[...] = jnp.where(m <= 0.0, -jnp.inf, s)


def _topk_kernel(sa_ref, sb_ref, idx_ref, tanh_ref, scr_ref):
    scr_ref[...] = sa_ref[...]
    rmax0 = jnp.max(sb_ref[...], axis=0, keepdims=True)
    lane_r = jax.lax.broadcasted_iota(jnp.int32, (1, P_ROWS), 1)
    lane_c = jax.lax.broadcasted_iota(jnp.int32, (1, P_COLS), 1)
    lane = jax.lax.broadcasted_iota(jnp.int32, (1, K), 1)
    big = jnp.int32(2**30)

    def body(j, carry):
        rmax, vacc, iacc = carry
        m = jnp.max(rmax)
        c = jnp.min(jnp.where(rmax == m, lane_r, big))
        row = scr_ref[pl.ds(c, 1), :]
        l = jnp.min(jnp.where(row == m, lane_c, big))
        nrow = jnp.where(lane_c == l, -jnp.inf, row)
        scr_ref[pl.ds(c, 1), :] = nrow
        rmax = jnp.where(lane_r == c, jnp.max(nrow), rmax)
        vacc = jnp.where(lane == j, m, vacc)
        iacc = jnp.where(lane == j, c * P_COLS + l, iacc)
        return rmax, vacc, iacc

    init = (
        rmax0,
        jnp.full((1, K), -jnp.inf, jnp.float32),
        jnp.zeros((1, K), jnp.int32),
    )
    _, vals, idx = jax.lax.fori_loop(0, K, body, init)
    valid = vals > -jnp.inf
    nv = jnp.sum(valid.astype(jnp.int32))
    pos = jnp.maximum(nv - 1, 0)
    lane = jax.lax.broadcasted_iota(jnp.int32, (1, K), 1)
    last_idx = jnp.sum(jnp.where(lane == pos, idx, 0))
    last_val = jnp.sum(jnp.where(lane == pos, vals, 0.0))
    idx_ref[...] = jnp.where(valid, idx, last_idx)
    tanh_ref[...] = jnp.tanh(jnp.where(valid, vals, last_val))


def _gather_kernel(idx_ref, f_ref, t_ref, o_ref):
    i = pl.program_id(0)
    lane = jax.lax.broadcasted_iota(jnp.int32, (1, K), 1)
    t = jnp.sum(jnp.where(lane == i, t_ref[...], 0.0))
    o_ref[...] = f_ref[...] * t


@jax.jit
def kernel(feats, masks, W):
    m2d = masks.reshape(N_NODES, 1)

    scores = pl.pallas_call(
        _score_kernel,
        grid=(N_BLOCKS,),
        in_specs=[
            pl.BlockSpec((ROW_BLOCK, N_FEATS), lambda i: (i, 0)),
            pl.BlockSpec((N_FEATS, 1), lambda i: (0, 0)),
            pl.BlockSpec((ROW_BLOCK, 1), lambda i: (i, 0)),
        ],
        out_specs=pl.BlockSpec((ROW_BLOCK, 1), lambda i: (i, 0)),
        out_shape=jax.ShapeDtypeStruct((N_NODES, 1), jnp.float32),
    )(feats, W, m2d)

    s = jnp.concatenate(
        [
            scores.reshape(N_NODES),
            jnp.full((N_PADDED - N_NODES,), -jnp.inf, jnp.float32),
        ]
    )
    sa = s.reshape(P_ROWS, P_COLS)
    sb = sa.T

    fidx, tval = pl.pallas_call(
        _topk_kernel,
        in_specs=[
            pl.BlockSpec((P_ROWS, P_COLS), lambda: (0, 0)),
            pl.BlockSpec((P_COLS, P_ROWS), lambda: (0, 0)),
        ],
        out_specs=[
            pl.BlockSpec((1, K), lambda: (0, 0)),
            pl.BlockSpec((1, K), lambda: (0, 0)),
        ],
        out_shape=[
            jax.ShapeDtypeStruct((1, K), jnp.int32),
            jax.ShapeDtypeStruct((1, K), jnp.float32),
        ],
        scratch_shapes=[pltpu.VMEM((P_ROWS, P_COLS), jnp.float32)],
    )(sa, sb)

    grid_spec = pltpu.PrefetchScalarGridSpec(
        num_scalar_prefetch=1,
        grid=(K,),
        in_specs=[
            pl.BlockSpec((1, 1, N_FEATS), lambda i, idx: (idx[i], 0, 0)),
            pl.BlockSpec((1, K), lambda i, idx: (0, 0)),
        ],
        out_specs=pl.BlockSpec((1, 1, N_FEATS), lambda i, idx: (i, 0, 0)),
    )

    selects = pl.pallas_call(
        _gather_kernel,
        grid_spec=grid_spec,
        out_shape=jax.ShapeDtypeStruct((K, 1, N_FEATS), jnp.float32),
    )(fidx.reshape(K), feats.reshape(N_NODES, 1, N_FEATS), tval)

    return selects.reshape(K, N_FEATS)


# speedup vs baseline: 1.4354x; 1.4354x over previous
import functools

import jax
import jax.numpy as jnp
from jax.experimental import pallas as pl
from jax.experimental.pallas import tpu as pltpu

N_NODES = 50000
N_FEATS = 256
K = 256
ROW_BLOCK = 2000
N_BLOCKS = N_NODES // ROW_BLOCK
P_ROWS = 512
P_COLS = 128
N_PADDED = P_ROWS * P_COLS


def _score_kernel(f_ref, w_ref, m_ref, o_ref):
    w = w_ref[...]
    nrm = jnp.sqrt(jnp.sum(w * w))
    s = jnp.dot(f_ref[...], w, preferred_element_type=jnp.float32)
    s = s / nrm
    m = m_ref[...]
    o_ref[...] = jnp.where(m <= 0.0, -jnp.inf, s)


def _topk_kernel(s_ref, idx_ref, tanh_ref):
    s0 = s_ref[...]
    R, C = s0.shape
    flat = (
        jax.lax.broadcasted_iota(jnp.int32, (R, C), 0) * C
        + jax.lax.broadcasted_iota(jnp.int32, (R, C), 1)
    )
    lane = jax.lax.broadcasted_iota(jnp.int32, (1, K), 1)
    big = jnp.int32(2**30)

    def body(j, carry):
        s, vacc, iacc = carry
        m = jnp.max(s)
        fm = jnp.min(jnp.where(s == m, flat, big))
        s = jnp.where(flat == fm, -jnp.inf, s)
        vacc = jnp.where(lane == j, m, vacc)
        iacc = jnp.where(lane == j, fm, iacc)
        return s, vacc, iacc

    init = (
        s0,
        jnp.full((1, K), -jnp.inf, jnp.float32),
        jnp.zeros((1, K), jnp.int32),
    )
    _, vals, idx = jax.lax.fori_loop(0, K, body, init)
    valid = vals > -jnp.inf
    nv = jnp.sum(valid.astype(jnp.int32))
    pos = jnp.maximum(nv - 1, 0)
    lane = jax.lax.broadcasted_iota(jnp.int32, (1, K), 1)
    last_idx = jnp.sum(jnp.where(lane == pos, idx, 0))
    last_val = jnp.sum(jnp.where(lane == pos, vals, 0.0))
    idx_ref[...] = jnp.where(valid, idx, last_idx)
    tanh_ref[...] = jnp.tanh(jnp.where(valid, vals, last_val))


GB = 8


def _gather_kernel(idx_ref, *refs):
    f_refs = refs[:GB]
    t_ref = refs[GB]
    o_ref = refs[GB + 1]
    i = pl.program_id(0)
    lane = jax.lax.broadcasted_iota(jnp.int32, (1, K), 1)
    tv = t_ref[...]
    for j in range(GB):
        t = jnp.sum(jnp.where(lane == i * GB + j, tv, 0.0))
        o_ref[j, :] = f_refs[j][0, 0, :] * t


@jax.jit
def kernel(feats, masks, W):
    m2d = masks.reshape(N_NODES, 1)

    scores = pl.pallas_call(
        _score_kernel,
        grid=(N_BLOCKS,),
        in_specs=[
            pl.BlockSpec((ROW_BLOCK, N_FEATS), lambda i: (i, 0)),
            pl.BlockSpec((N_FEATS, 1), lambda i: (0, 0)),
            pl.BlockSpec((ROW_BLOCK, 1), lambda i: (i, 0)),
        ],
        out_specs=pl.BlockSpec((ROW_BLOCK, 1), lambda i: (i, 0)),
        out_shape=jax.ShapeDtypeStruct((N_NODES, 1), jnp.float32),
    )(feats, W, m2d)

    s = jnp.concatenate(
        [
            scores.reshape(N_NODES),
            jnp.full((N_PADDED - N_NODES,), -jnp.inf, jnp.float32),
        ]
    )
    sa = s.reshape(8, N_PADDED // 8)

    fidx, tval = pl.pallas_call(
        _topk_kernel,
        in_specs=[pl.BlockSpec((8, N_PADDED // 8), lambda: (0, 0))],
        out_specs=[
            pl.BlockSpec((1, K), lambda: (0, 0)),
            pl.BlockSpec((1, K), lambda: (0, 0)),
        ],
        out_shape=[
            jax.ShapeDtypeStruct((1, K), jnp.int32),
            jax.ShapeDtypeStruct((1, K), jnp.float32),
        ],
    )(sa)

    def _row_spec(j):
        return pl.BlockSpec(
            (1, 1, N_FEATS), lambda i, idx, j=j: (idx[i * GB + j], 0, 0)
        )

    grid_spec = pltpu.PrefetchScalarGridSpec(
        num_scalar_prefetch=1,
        grid=(K // GB,),
        in_specs=[_row_spec(j) for j in range(GB)]
        + [pl.BlockSpec((1, K), lambda i, idx: (0, 0))],
        out_specs=pl.BlockSpec((GB, N_FEATS), lambda i, idx: (i, 0)),
    )

    f3 = feats.reshape(N_NODES, 1, N_FEATS)
    selects = pl.pallas_call(
        _gather_kernel,
        grid_spec=grid_spec,
        out_shape=jax.ShapeDtypeStruct((K, N_FEATS), jnp.float32),
    )(fidx.reshape(K), *([f3] * GB), tval)

    return selects


# speedup vs baseline: 1.4490x; 1.0095x over previous
import functools

import jax
import jax.numpy as jnp
from jax.experimental import pallas as pl
from jax.experimental.pallas import tpu as pltpu

N_NODES = 50000
N_FEATS = 256
K = 256
ROW_BLOCK = 2000
N_BLOCKS = N_NODES // ROW_BLOCK
P_ROWS = 512
P_COLS = 128
N_PADDED = P_ROWS * P_COLS


def _score_kernel(f_ref, w_ref, m_ref, o_ref):
    w = w_ref[...]
    nrm = jnp.sqrt(jnp.sum(w * w))
    s = jnp.dot(f_ref[...], w, preferred_element_type=jnp.float32)
    s = s / nrm
    m = m_ref[...]
    o_ref[...] = jnp.where(m <= 0.0, -jnp.inf, s)


def _topk_kernel(s_ref, idx_ref, tanh_ref):
    s0 = s_ref[...]
    R, C = s0.shape
    flat = (
        jax.lax.broadcasted_iota(jnp.int32, (R, C), 0) * C
        + jax.lax.broadcasted_iota(jnp.int32, (R, C), 1)
    )
    lane = jax.lax.broadcasted_iota(jnp.int32, (1, K), 1)
    big = jnp.int32(2**30)

    UNROLL = 4

    def body(j, carry):
        s, vacc, iacc = carry
        for u in range(UNROLL):
            m = jnp.max(s)
            fm = jnp.min(jnp.where(s == m, flat, big))
            s = jnp.where(flat == fm, -jnp.inf, s)
            vacc = jnp.where(lane == j * UNROLL + u, m, vacc)
            iacc = jnp.where(lane == j * UNROLL + u, fm, iacc)
        return s, vacc, iacc

    init = (
        s0,
        jnp.full((1, K), -jnp.inf, jnp.float32),
        jnp.zeros((1, K), jnp.int32),
    )
    _, vals, idx = jax.lax.fori_loop(0, K // UNROLL, body, init)
    valid = vals > -jnp.inf
    nv = jnp.sum(valid.astype(jnp.int32))
    pos = jnp.maximum(nv - 1, 0)
    lane = jax.lax.broadcasted_iota(jnp.int32, (1, K), 1)
    last_idx = jnp.sum(jnp.where(lane == pos, idx, 0))
    last_val = jnp.sum(jnp.where(lane == pos, vals, 0.0))
    idx_ref[...] = jnp.where(valid, idx, last_idx)
    tanh_ref[...] = jnp.tanh(jnp.where(valid, vals, last_val))


GB = 8


def _gather_kernel(idx_ref, *refs):
    f_refs = refs[:GB]
    t_ref = refs[GB]
    o_ref = refs[GB + 1]
    i = pl.program_id(0)
    lane = jax.lax.broadcasted_iota(jnp.int32, (1, K), 1)
    tv = t_ref[...]
    for j in range(GB):
        t = jnp.sum(jnp.where(lane == i * GB + j, tv, 0.0))
        o_ref[j, :] = f_refs[j][0, 0, :] * t


@jax.jit
def kernel(feats, masks, W):
    m2d = masks.reshape(N_NODES, 1)

    scores = pl.pallas_call(
        _score_kernel,
        grid=(N_BLOCKS,),
        in_specs=[
            pl.BlockSpec((ROW_BLOCK, N_FEATS), lambda i: (i, 0)),
            pl.BlockSpec((N_FEATS, 1), lambda i: (0, 0)),
            pl.BlockSpec((ROW_BLOCK, 1), lambda i: (i, 0)),
        ],
        out_specs=pl.BlockSpec((ROW_BLOCK, 1), lambda i: (i, 0)),
        out_shape=jax.ShapeDtypeStruct((N_NODES, 1), jnp.float32),
    )(feats, W, m2d)

    s = jnp.concatenate(
        [
            scores.reshape(N_NODES),
            jnp.full((N_PADDED - N_NODES,), -jnp.inf, jnp.float32),
        ]
    )
    sa = s.reshape(8, N_PADDED // 8)

    fidx, tval = pl.pallas_call(
        _topk_kernel,
        in_specs=[pl.BlockSpec((8, N_PADDED // 8), lambda: (0, 0))],
        out_specs=[
            pl.BlockSpec((1, K), lambda: (0, 0)),
            pl.BlockSpec((1, K), lambda: (0, 0)),
        ],
        out_shape=[
            jax.ShapeDtypeStruct((1, K), jnp.int32),
            jax.ShapeDtypeStruct((1, K), jnp.float32),
        ],
    )(sa)

    def _row_spec(j):
        return pl.BlockSpec(
            (1, 1, N_FEATS), lambda i, idx, j=j: (idx[i * GB + j], 0, 0)
        )

    grid_spec = pltpu.PrefetchScalarGridSpec(
        num_scalar_prefetch=1,
        grid=(K // GB,),
        in_specs=[_row_spec(j) for j in range(GB)]
        + [pl.BlockSpec((1, K), lambda i, idx: (0, 0))],
        out_specs=pl.BlockSpec((GB, N_FEATS), lambda i, idx: (i, 0)),
    )

    f3 = feats.reshape(N_NODES, 1, N_FEATS)
    selects = pl.pallas_call(
        _gather_kernel,
        grid_spec=grid_spec,
        out_shape=jax.ShapeDtypeStruct((K, N_FEATS), jnp.float32),
    )(fidx.reshape(K), *([f3] * GB), tval)

    return selects
